# baseline (speedup 1.0000x reference)
import functools

import jax
import jax.numpy as jnp
from jax import lax
from jax.experimental import pallas as pl
from jax.experimental.pallas import tpu as pltpu
from jax.experimental.pallas import tpu_sc as plsc

N = 4096
D = 128
EK = 262144
EHK = 131072
NC = 2
NS = 16
NW = NC * NS
B = 64
CK = EK // NW // B
CHK = EHK // NW // B
RPT = N // NS

_BR = 512


def _prea_body(sH_ref, sK_ref, g_ref, Xn_ref):
    g_ref[...] = jnp.tanh(sH_ref[...])
    X = sK_ref[...]
    norm = jnp.sqrt(jnp.sum(X * X, axis=1, keepdims=True))
    Xn_ref[...] = X / norm


def _prea(state_H, state_K):
    return pl.pallas_call(
        _prea_body,
        out_shape=[
            jax.ShapeDtypeStruct((1, N), jnp.float32),
            jax.ShapeDtypeStruct((N, D), jnp.float32),
        ],
    )(state_H[None, :], state_K)


def _preb_body(sH_ref, bH_ref, W_ref, h0_ref):
    step = pl.program_id(0)
    g = jnp.tanh(sH_ref[...])

    @pl.when(step == 0)
    def _init():
        h0_ref[...] = -sH_ref[...] + bH_ref[...]

    Wb = W_ref[...]
    wg = jnp.dot(Wb, g.T, preferred_element_type=jnp.float32)
    gs = jnp.tanh(sH_ref[:, pl.ds(step * _BR, _BR)])
    wtg = jnp.dot(gs, Wb, preferred_element_type=jnp.float32)
    h0_ref[...] += 0.5 * wtg
    h0_ref[:, pl.ds(step * _BR, _BR)] += 0.5 * wg.T


def _preb(state_H, bias_H, weights_H):
    return pl.pallas_call(
        _preb_body,
        grid=(N // _BR,),
        in_specs=[
            pl.BlockSpec((1, N), lambda i: (0, 0)),
            pl.BlockSpec((1, N), lambda i: (0, 0)),
            pl.BlockSpec((_BR, N), lambda i: (i, 0)),
        ],
        out_specs=[pl.BlockSpec((1, N), lambda i: (0, 0))],
        out_shape=[jax.ShapeDtypeStruct((1, N), jnp.float32)],
    )(state_H[None, :], bias_H[None, :], weights_H)


_mesh = plsc.VectorSubcoreMesh(core_axis_name="c", subcore_axis_name="s")


def _splat(ref1d, e):
    return plsc.load_gather(ref1d, [jnp.full((16,), e, jnp.int32)])


@functools.partial(
    pl.kernel,
    out_type=[
        jax.ShapeDtypeStruct((NC, N, D), jnp.float32),
        jax.ShapeDtypeStruct((NW, N), jnp.float32),
    ],
    mesh=_mesh,
    compiler_params=pltpu.CompilerParams(needs_layout_passes=False),
    scratch_types=[
        pltpu.VMEM((B,), jnp.int32),
        pltpu.VMEM((B,), jnp.int32),
        pltpu.VMEM((B,), jnp.int32),
        pltpu.VMEM((B,), jnp.int32),
        pltpu.VMEM((B,), jnp.int32),
        pltpu.VMEM((B,), jnp.int32),
        pltpu.VMEM((B,), jnp.int32),
        pltpu.VMEM((B,), jnp.int32),
        pltpu.VMEM((B,), jnp.float32),
        pltpu.VMEM((B,), jnp.float32),
        pltpu.VMEM((B,), jnp.float32),
        pltpu.VMEM((B,), jnp.float32),
        pltpu.VMEM((B, D), jnp.float32),
        pltpu.VMEM((B, D), jnp.float32),
        pltpu.VMEM((B, D), jnp.float32),
        pltpu.VMEM((B, D), jnp.float32),
        pltpu.VMEM((N,), jnp.float32),
        pltpu.VMEM((N,), jnp.float32),
        pltpu.VMEM_SHARED((N, D), jnp.float32),
        pltpu.VMEM_SHARED((N, D), jnp.float32),
        pltpu.SemaphoreType.DMA,
        pltpu.SemaphoreType.DMA,
        pltpu.SemaphoreType.DMA,
        pltpu.SemaphoreType.DMA,
        pltpu.SemaphoreType.DMA,
        pltpu.SemaphoreType.DMA,
    ],
)
def _edges(xn, g, ik0, ik1, wk, ih0, ih1, whk, whh, fko, fho,
           gi0, gj0, gi1, gj1, si0, sj0, si1, sj1,
           wa0, wa1, wb0, wb1,
           xi0, xj0, xi1, xj1, gall, fh, fk_acc, xn_s,
           sg0, sg1, ss0, ss1, si0s, si1s):
    cid = lax.axis_index("c")
    sid = lax.axis_index("s")
    wid = cid * NS + sid
    z16 = jnp.zeros((16,), jnp.float32)
    lane0 = jnp.arange(16, dtype=jnp.int32) == 0
    GI = (gi0, gi1)
    GJ = (gj0, gj1)
    SI = (si0, si1)
    SJ = (sj0, sj1)
    WA = (wa0, wa1)
    WB = (wb0, wb1)
    XI = (xi0, xi1)
    XJ = (xj0, xj1)
    SG = (sg0, sg1)
    SS = (ss0, ss1)
    SM = (si0s, si1s)

    pltpu.sync_copy(g, gall)
    pltpu.sync_copy(xn.at[pl.ds(sid * RPT, RPT)], xn_s.at[pl.ds(sid * RPT, RPT)])

    @plsc.parallel_loop(0, B)
    def _zrow(r):
        for k in range(D // 16):
            xi0[r, pl.ds(k * 16, 16)] = z16

    @plsc.parallel_loop(0, N // 16)
    def _zfh(i):
        fh[pl.ds(i * 16, 16)] = z16

    for r0 in range(0, RPT, B):
        pltpu.sync_copy(xi0, fk_acc.at[pl.ds(sid * RPT + r0, B)])
    plsc.subcore_barrier()

    def _phase(base0, C, i0_h, i1_h, wa_h, wb_h, hk):
        def _idx_dma(n, b):
            off = base0 + n * B
            pltpu.async_copy(i0_h.at[pl.ds(off, B)], GI[b], SM[b])
            pltpu.async_copy(i1_h.at[pl.ds(off, B)], GJ[b], SM[b])
            pltpu.async_copy(wa_h.at[pl.ds(off, B)], WA[b], SM[b])
            if hk:
                pltpu.async_copy(wb_h.at[pl.ds(off, B)], WB[b], SM[b])

        def _drain_idx(n, b):
            off = base0 + n * B
            pltpu.make_async_copy(i0_h.at[pl.ds(off, B)], GI[b], SM[b]).wait()
            pltpu.make_async_copy(i1_h.at[pl.ds(off, B)], GJ[b], SM[b]).wait()
            pltpu.make_async_copy(wa_h.at[pl.ds(off, B)], WA[b], SM[b]).wait()
            if hk:
                pltpu.make_async_copy(wb_h.at[pl.ds(off, B)], WB[b], SM[b]).wait()

        def _gather(b):
            pltpu.async_copy(xn_s.at[GI[b]], XI[b], SG[b])
            pltpu.async_copy(xn_s.at[GJ[b]], XJ[b], SG[b])

        def _drain_gather(b):
            pltpu.make_async_copy(xn_s.at[GI[b]], XI[b], SG[b]).wait()
            pltpu.make_async_copy(xn_s.at[GJ[b]], XJ[b], SG[b]).wait()

        def _scatter(b):
            pltpu.async_copy(XJ[b], fk_acc.at[SI[b]], SS[b], add=True)
            pltpu.async_copy(XI[b], fk_acc.at[SJ[b]], SS[b], add=True)

        def _drain_scatter(b):
            pltpu.make_async_copy(XJ[b], fk_acc.at[SI[b]], SS[b]).wait()
            pltpu.make_async_copy(XI[b], fk_acc.at[SJ[b]], SS[b]).wait()

        def _compute(b):
            xi_b, xj_b = XI[b], XJ[b]
            wa_b, wb_b = WA[b], WB[b]
            gi_b, gj_b = GI[b], GJ[b]

            @plsc.parallel_loop(0, B, unroll=8)
            def _edge(e):
                xiv = [xi_b[e, pl.ds(k * 16, 16)] for k in range(D // 16)]
                xjv = [xj_b[e, pl.ds(k * 16, 16)] for k in range(D // 16)]
                acc = xiv[0] * xjv[0]
                for k in range(1, D // 16):
                    acc = acc + xiv[k] * xjv[k]
                sv = jnp.broadcast_to(jnp.sum(acc), (16,))
                if hk:
                    iiv = _splat(gi_b, e)
                    jjv = _splat(gj_b, e)
                    giv = plsc.load_gather(gall, [iiv])
                    gjv = plsc.load_gather(gall, [jjv])
                    cv = _splat(wb_b, e) * sv
                    plsc.addupdate_scatter(fh, [iiv], cv * gjv, mask=lane0)
                    plsc.addupdate_scatter(fh, [jjv], cv * giv, mask=lane0)
                    qv = -(giv * gjv * _splat(wa_b, e))
                else:
                    qv = -_splat(wa_b, e) * sv
                for k in range(D // 16):
                    xj_b[e, pl.ds(k * 16, 16)] = qv * xjv[k]
                    xi_b[e, pl.ds(k * 16, 16)] = qv * xiv[k]

        def _save_idx(b):
            for k in range(B // 16):
                s = pl.ds(k * 16, 16)
                SI[b][s] = GI[b][s]
                SJ[b][s] = GJ[b][s]

        _idx_dma(0, 0)
        _idx_dma(1, 1)
        _drain_idx(0, 0)
        _gather(0)

        @pl.loop(0, C, step=2)
        def _loop(c):
            for b in (0, 1):
                o = 1 - b
                n = c + b
                if b == 0:
                    @pl.when(c > 0)
                    def _():
                        _drain_scatter(o)
                    _drain_idx(n + 1, o)
                    _gather(o)
                else:
                    _drain_scatter(o)

                    @pl.when(c < C - 2)
                    def _():
                        _drain_idx(n + 1, o)
                        _gather(o)
                _drain_gather(b)
                _save_idx(b)
                _compute(b)

                @pl.when(c < C - 2)
                def _():
                    _idx_dma(n + 2, b)
                _scatter(b)

        _drain_scatter((C - 1) % 2)

    _phase(wid * (CK * B), CK, ik0, ik1, wk, wk, False)
    _phase(wid * (CHK * B), CHK, ih0, ih1, whk, whh, True)
    plsc.subcore_barrier()

    pltpu.sync_copy(fk_acc.at[pl.ds(sid * RPT, RPT)],
                    fko.at[cid, pl.ds(sid * RPT, RPT)])
    pltpu.sync_copy(fh, fho.at[wid])


def _post_body(Xn_ref, fkp_ref, fhp_ref, h0_ref, fH_ref, fK_ref):
    fH_ref[...] = h0_ref[...] + jnp.sum(fhp_ref[...], axis=0, keepdims=True)
    X = Xn_ref[...]
    fKp = fkp_ref[0] + fkp_ref[1]
    proj = jnp.sum(X * fKp, axis=1, keepdims=True)
    fK_ref[...] = -fKp + X * proj


def _post(Xn, fkp, fhp, h0):
    return pl.pallas_call(
        _post_body,
        out_shape=[
            jax.ShapeDtypeStruct((1, N), jnp.float32),
            jax.ShapeDtypeStruct((N, D), jnp.float32),
        ],
    )(Xn, fkp, fhp, h0)


def kernel(t, state_H, state_K, ind_K, ind_HK, kappa_K, kappa_H, weights_H, bias_H, weights_HK, weights_K):
    g2, Xn = _prea(state_H, state_K)
    (h0,) = _preb(state_H, bias_H, weights_H)
    g = g2.reshape(N)
    whh = weights_HK[:, 0] / kappa_H
    whk = weights_HK[:, 0] / kappa_K
    fkp, fhp = _edges(Xn, g, ind_K[:, 0], ind_K[:, 1], weights_K,
                      ind_HK[:, 0], ind_HK[:, 1], whk, whh)
    fH2, fK = _post(Xn, fkp, fhp, h0)
    return (fH2.reshape(N), fK)


# speedup vs baseline: 1.1953x; 1.1953x over previous
import functools

import jax
import jax.numpy as jnp
from jax import lax
from jax.experimental import pallas as pl
from jax.experimental.pallas import tpu as pltpu
from jax.experimental.pallas import tpu_sc as plsc

N = 4096
D = 128
EK = 262144
EHK = 131072
NC = 2
NS = 16
NW = NC * NS
B = 128
CK = EK // NW // B
CHK = EHK // NW // B
RPT = N // NS

_BR = 512


def _prea_body(sH_ref, sK_ref, g_ref, Xn_ref):
    g_ref[...] = jnp.tanh(sH_ref[...])
    X = sK_ref[...]
    norm = jnp.sqrt(jnp.sum(X * X, axis=1, keepdims=True))
    Xn_ref[...] = X / norm


def _prea(state_H, state_K):
    return pl.pallas_call(
        _prea_body,
        out_shape=[
            jax.ShapeDtypeStruct((1, N), jnp.float32),
            jax.ShapeDtypeStruct((N, D), jnp.float32),
        ],
    )(state_H[None, :], state_K)


def _preb_body(sH_ref, bH_ref, W_ref, h0_ref):
    step = pl.program_id(0)
    g = jnp.tanh(sH_ref[...])

    @pl.when(step == 0)
    def _init():
        h0_ref[...] = -sH_ref[...] + bH_ref[...]

    Wb = W_ref[...]
    wg = jnp.dot(Wb, g.T, preferred_element_type=jnp.float32)
    gs = jnp.tanh(sH_ref[:, pl.ds(step * _BR, _BR)])
    wtg = jnp.dot(gs, Wb, preferred_element_type=jnp.float32)
    h0_ref[...] += 0.5 * wtg
    h0_ref[:, pl.ds(step * _BR, _BR)] += 0.5 * wg.T


def _preb(state_H, bias_H, weights_H):
    return pl.pallas_call(
        _preb_body,
        grid=(N // _BR,),
        in_specs=[
            pl.BlockSpec((1, N), lambda i: (0, 0)),
            pl.BlockSpec((1, N), lambda i: (0, 0)),
            pl.BlockSpec((_BR, N), lambda i: (i, 0)),
        ],
        out_specs=[pl.BlockSpec((1, N), lambda i: (0, 0))],
        out_shape=[jax.ShapeDtypeStruct((1, N), jnp.float32)],
    )(state_H[None, :], bias_H[None, :], weights_H)


_mesh = plsc.VectorSubcoreMesh(core_axis_name="c", subcore_axis_name="s")


def _splat(ref1d, e):
    return plsc.load_gather(ref1d, [jnp.full((16,), e, jnp.int32)])


@functools.partial(
    pl.kernel,
    out_type=[
        jax.ShapeDtypeStruct((NC, N, D), jnp.float32),
        jax.ShapeDtypeStruct((NW, N), jnp.float32),
    ],
    mesh=_mesh,
    compiler_params=pltpu.CompilerParams(needs_layout_passes=False),
    scratch_types=[
        pltpu.VMEM((B,), jnp.int32),
        pltpu.VMEM((B,), jnp.int32),
        pltpu.VMEM((B,), jnp.int32),
        pltpu.VMEM((B,), jnp.int32),
        pltpu.VMEM((B,), jnp.int32),
        pltpu.VMEM((B,), jnp.int32),
        pltpu.VMEM((B,), jnp.int32),
        pltpu.VMEM((B,), jnp.int32),
        pltpu.VMEM((B,), jnp.float32),
        pltpu.VMEM((B,), jnp.float32),
        pltpu.VMEM((B,), jnp.float32),
        pltpu.VMEM((B,), jnp.float32),
        pltpu.VMEM((B, D), jnp.float32),
        pltpu.VMEM((B, D), jnp.float32),
        pltpu.VMEM((B, D), jnp.float32),
        pltpu.VMEM((B, D), jnp.float32),
        pltpu.VMEM((N,), jnp.float32),
        pltpu.VMEM((N,), jnp.float32),
        pltpu.VMEM_SHARED((N, D), jnp.float32),
        pltpu.SemaphoreType.DMA,
        pltpu.SemaphoreType.DMA,
        pltpu.SemaphoreType.DMA,
        pltpu.SemaphoreType.DMA,
        pltpu.SemaphoreType.DMA,
        pltpu.SemaphoreType.DMA,
    ],
)
def _edges(xn, g, ik0, ik1, wk, ih0, ih1, whk, whh, fko, fho,
           gi0, gj0, gi1, gj1, si0, sj0, si1, sj1,
           wa0, wa1, wb0, wb1,
           xi0, xj0, xi1, xj1, gall, fh, fk_acc,
           sg0, sg1, ss0, ss1, si0s, si1s):
    cid = lax.axis_index("c")
    sid = lax.axis_index("s")
    wid = cid * NS + sid
    z16 = jnp.zeros((16,), jnp.float32)
    lane0 = jnp.arange(16, dtype=jnp.int32) == 0
    GI = (gi0, gi1)
    GJ = (gj0, gj1)
    SI = (si0, si1)
    SJ = (sj0, sj1)
    WA = (wa0, wa1)
    WB = (wb0, wb1)
    XI = (xi0, xi1)
    XJ = (xj0, xj1)
    SG = (sg0, sg1)
    SS = (ss0, ss1)
    SM = (si0s, si1s)

    pltpu.sync_copy(g, gall)

    @plsc.parallel_loop(0, B)
    def _zrow(r):
        for k in range(D // 16):
            xi0[r, pl.ds(k * 16, 16)] = z16

    @plsc.parallel_loop(0, N // 16)
    def _zfh(i):
        fh[pl.ds(i * 16, 16)] = z16

    for r0 in range(0, RPT, B):
        pltpu.sync_copy(xi0, fk_acc.at[pl.ds(sid * RPT + r0, B)])
    plsc.subcore_barrier()

    def _phase(base0, C, i0_h, i1_h, wa_h, wb_h, hk):
        def _idx_dma(n, b):
            off = base0 + n * B
            pltpu.async_copy(i0_h.at[pl.ds(off, B)], GI[b], SM[b])
            pltpu.async_copy(i1_h.at[pl.ds(off, B)], GJ[b], SM[b])
            pltpu.async_copy(wa_h.at[pl.ds(off, B)], WA[b], SM[b])
            if hk:
                pltpu.async_copy(wb_h.at[pl.ds(off, B)], WB[b], SM[b])

        def _drain_idx(n, b):
            off = base0 + n * B
            pltpu.make_async_copy(i0_h.at[pl.ds(off, B)], GI[b], SM[b]).wait()
            pltpu.make_async_copy(i1_h.at[pl.ds(off, B)], GJ[b], SM[b]).wait()
            pltpu.make_async_copy(wa_h.at[pl.ds(off, B)], WA[b], SM[b]).wait()
            if hk:
                pltpu.make_async_copy(wb_h.at[pl.ds(off, B)], WB[b], SM[b]).wait()

        def _gather(b):
            pltpu.async_copy(xn.at[GI[b]], XI[b], SG[b])
            pltpu.async_copy(xn.at[GJ[b]], XJ[b], SG[b])

        def _drain_gather(b):
            pltpu.make_async_copy(xn.at[GI[b]], XI[b], SG[b]).wait()
            pltpu.make_async_copy(xn.at[GJ[b]], XJ[b], SG[b]).wait()

        def _scatter(b):
            pltpu.async_copy(XJ[b], fk_acc.at[SI[b]], SS[b], add=True)
            pltpu.async_copy(XI[b], fk_acc.at[SJ[b]], SS[b], add=True)

        def _drain_scatter(b):
            pltpu.make_async_copy(XJ[b], fk_acc.at[SI[b]], SS[b]).wait()
            pltpu.make_async_copy(XI[b], fk_acc.at[SJ[b]], SS[b]).wait()

        def _compute(b):
            xi_b, xj_b = XI[b], XJ[b]
            wa_b, wb_b = WA[b], WB[b]
            gi_b, gj_b = GI[b], GJ[b]

            @plsc.parallel_loop(0, B, unroll=8)
            def _edge(e):
                xiv = [xi_b[e, pl.ds(k * 16, 16)] for k in range(D // 16)]
                xjv = [xj_b[e, pl.ds(k * 16, 16)] for k in range(D // 16)]
                acc = xiv[0] * xjv[0]
                for k in range(1, D // 16):
                    acc = acc + xiv[k] * xjv[k]
                sv = jnp.broadcast_to(jnp.sum(acc), (16,))
                if hk:
                    iiv = _splat(gi_b, e)
                    jjv = _splat(gj_b, e)
                    giv = plsc.load_gather(gall, [iiv])
                    gjv = plsc.load_gather(gall, [jjv])
                    cv = _splat(wb_b, e) * sv
                    plsc.addupdate_scatter(fh, [iiv], cv * gjv, mask=lane0)
                    plsc.addupdate_scatter(fh, [jjv], cv * giv, mask=lane0)
                    qv = -(giv * gjv * _splat(wa_b, e))
                else:
                    qv = -_splat(wa_b, e) * sv
                for k in range(D // 16):
                    xj_b[e, pl.ds(k * 16, 16)] = qv * xjv[k]
                    xi_b[e, pl.ds(k * 16, 16)] = qv * xiv[k]

        def _save_idx(b):
            for k in range(B // 16):
                s = pl.ds(k * 16, 16)
                SI[b][s] = GI[b][s]
                SJ[b][s] = GJ[b][s]

        _idx_dma(0, 0)
        _idx_dma(1, 1)
        _drain_idx(0, 0)
        _gather(0)

        @pl.loop(0, C, step=2)
        def _loop(c):
            for b in (0, 1):
                o = 1 - b
                n = c + b
                if b == 0:
                    @pl.when(c > 0)
                    def _():
                        _drain_scatter(o)
                    _drain_idx(n + 1, o)
                    _gather(o)
                else:
                    _drain_scatter(o)

                    @pl.when(c < C - 2)
                    def _():
                        _drain_idx(n + 1, o)
                        _gather(o)
                _drain_gather(b)
                _save_idx(b)
                _compute(b)

                @pl.when(c < C - 2)
                def _():
                    _idx_dma(n + 2, b)
                _scatter(b)

        _drain_scatter((C - 1) % 2)

    _phase(wid * (CK * B), CK, ik0, ik1, wk, wk, False)
    _phase(wid * (CHK * B), CHK, ih0, ih1, whk, whh, True)
    plsc.subcore_barrier()

    pltpu.sync_copy(fk_acc.at[pl.ds(sid * RPT, RPT)],
                    fko.at[cid, pl.ds(sid * RPT, RPT)])
    pltpu.sync_copy(fh, fho.at[wid])


def _post_body(Xn_ref, fkp_ref, fhp_ref, h0_ref, fH_ref, fK_ref):
    fH_ref[...] = h0_ref[...] + jnp.sum(fhp_ref[...], axis=0, keepdims=True)
    X = Xn_ref[...]
    fKp = fkp_ref[0] + fkp_ref[1]
    proj = jnp.sum(X * fKp, axis=1, keepdims=True)
    fK_ref[...] = -fKp + X * proj


def _post(Xn, fkp, fhp, h0):
    return pl.pallas_call(
        _post_body,
        out_shape=[
            jax.ShapeDtypeStruct((1, N), jnp.float32),
            jax.ShapeDtypeStruct((N, D), jnp.float32),
        ],
    )(Xn, fkp, fhp, h0)


def kernel(t, state_H, state_K, ind_K, ind_HK, kappa_K, kappa_H, weights_H, bias_H, weights_HK, weights_K):
    g2, Xn = _prea(state_H, state_K)
    (h0,) = _preb(state_H, bias_H, weights_H)
    g = g2.reshape(N)
    whh = weights_HK[:, 0] / kappa_H
    whk = weights_HK[:, 0] / kappa_K
    fkp, fhp = _edges(Xn, g, ind_K[:, 0], ind_K[:, 1], weights_K,
                      ind_HK[:, 0], ind_HK[:, 1], whk, whh)
    fH2, fK = _post(Xn, fkp, fhp, h0)
    return (fH2.reshape(N), fK)


# speedup vs baseline: 1.2020x; 1.0057x over previous
import functools

import jax
import jax.numpy as jnp
from jax import lax
from jax.experimental import pallas as pl
from jax.experimental.pallas import tpu as pltpu
from jax.experimental.pallas import tpu_sc as plsc

N = 4096
D = 128
EK = 262144
EHK = 131072
NC = 2
NS = 16
NW = NC * NS
B = 128
CK = EK // NW // B
CHK = EHK // NW // B
RPT = N // NS

_BR = 512


def _prea_body(sH_ref, sK_ref, g_ref, Xn_ref):
    g_ref[...] = jnp.tanh(sH_ref[...])
    X = sK_ref[...]
    norm = jnp.sqrt(jnp.sum(X * X, axis=1, keepdims=True))
    Xn_ref[...] = X / norm


def _prea(state_H, state_K):
    return pl.pallas_call(
        _prea_body,
        out_shape=[
            jax.ShapeDtypeStruct((1, N), jnp.float32),
            jax.ShapeDtypeStruct((N, D), jnp.float32),
        ],
    )(state_H[None, :], state_K)


def _preb_body(sH_ref, bH_ref, W_ref, h0_ref):
    step = pl.program_id(0)
    g = jnp.tanh(sH_ref[...])

    @pl.when(step == 0)
    def _init():
        h0_ref[...] = -sH_ref[...] + bH_ref[...]

    Wb = W_ref[...]
    wg = jnp.dot(Wb, g.T, preferred_element_type=jnp.float32)
    gs = jnp.tanh(sH_ref[:, pl.ds(step * _BR, _BR)])
    wtg = jnp.dot(gs, Wb, preferred_element_type=jnp.float32)
    h0_ref[...] += 0.5 * wtg
    h0_ref[:, pl.ds(step * _BR, _BR)] += 0.5 * wg.T


def _preb(state_H, bias_H, weights_H):
    return pl.pallas_call(
        _preb_body,
        grid=(N // _BR,),
        in_specs=[
            pl.BlockSpec((1, N), lambda i: (0, 0)),
            pl.BlockSpec((1, N), lambda i: (0, 0)),
            pl.BlockSpec((_BR, N), lambda i: (i, 0)),
        ],
        out_specs=[pl.BlockSpec((1, N), lambda i: (0, 0))],
        out_shape=[jax.ShapeDtypeStruct((1, N), jnp.float32)],
    )(state_H[None, :], bias_H[None, :], weights_H)


_mesh = plsc.VectorSubcoreMesh(core_axis_name="c", subcore_axis_name="s")


def _splat(ref1d, e):
    return plsc.load_gather(ref1d, [jnp.full((16,), e, jnp.int32)])


@functools.partial(
    pl.kernel,
    out_type=[
        jax.ShapeDtypeStruct((NC, N, D), jnp.float32),
        jax.ShapeDtypeStruct((NW, N), jnp.float32),
    ],
    mesh=_mesh,
    compiler_params=pltpu.CompilerParams(needs_layout_passes=False),
    scratch_types=[
        pltpu.VMEM((B,), jnp.int32),
        pltpu.VMEM((B,), jnp.int32),
        pltpu.VMEM((B,), jnp.int32),
        pltpu.VMEM((B,), jnp.int32),
        pltpu.VMEM((B,), jnp.int32),
        pltpu.VMEM((B,), jnp.int32),
        pltpu.VMEM((B,), jnp.int32),
        pltpu.VMEM((B,), jnp.int32),
        pltpu.VMEM((B,), jnp.float32),
        pltpu.VMEM((B,), jnp.float32),
        pltpu.VMEM((B,), jnp.float32),
        pltpu.VMEM((B,), jnp.float32),
        pltpu.VMEM((B, D), jnp.float32),
        pltpu.VMEM((B, D), jnp.float32),
        pltpu.VMEM((B, D), jnp.float32),
        pltpu.VMEM((B, D), jnp.float32),
        pltpu.VMEM((N,), jnp.float32),
        pltpu.VMEM((N,), jnp.float32),
        pltpu.VMEM_SHARED((N, D), jnp.float32),
        pltpu.SemaphoreType.DMA,
        pltpu.SemaphoreType.DMA,
        pltpu.SemaphoreType.DMA,
        pltpu.SemaphoreType.DMA,
        pltpu.SemaphoreType.DMA,
        pltpu.SemaphoreType.DMA,
    ],
)
def _edges(xn, g, ik0, ik1, wk, ih0, ih1, whk, whh, fko, fho,
           gi0, gj0, gi1, gj1, si0, sj0, si1, sj1,
           wa0, wa1, wb0, wb1,
           xi0, xj0, xi1, xj1, gall, fh, fk_acc,
           sg0, sg1, ss0, ss1, si0s, si1s):
    cid = lax.axis_index("c")
    sid = lax.axis_index("s")
    wid = cid * NS + sid
    z16 = jnp.zeros((16,), jnp.float32)
    lane0 = jnp.arange(16, dtype=jnp.int32) == 0
    GI = (gi0, gi1)
    GJ = (gj0, gj1)
    SI = (si0, si1)
    SJ = (sj0, sj1)
    WA = (wa0, wa1)
    WB = (wb0, wb1)
    XI = (xi0, xi1)
    XJ = (xj0, xj1)
    SG = (sg0, sg1)
    SS = (ss0, ss1)
    SM = (si0s, si1s)

    pltpu.sync_copy(g, gall)

    @plsc.parallel_loop(0, B)
    def _zrow(r):
        for k in range(D // 16):
            xi0[r, pl.ds(k * 16, 16)] = z16

    @plsc.parallel_loop(0, N // 16)
    def _zfh(i):
        fh[pl.ds(i * 16, 16)] = z16

    for r0 in range(0, RPT, B):
        pltpu.sync_copy(xi0, fk_acc.at[pl.ds(sid * RPT + r0, B)])
    plsc.subcore_barrier()

    def _phase(base0, C, i0_h, i1_h, wa_h, wb_h, hk):
        def _idx_dma(n, b):
            off = base0 + n * B
            pltpu.async_copy(i0_h.at[pl.ds(off, B)], GI[b], SM[b])
            pltpu.async_copy(i1_h.at[pl.ds(off, B)], GJ[b], SM[b])
            pltpu.async_copy(wa_h.at[pl.ds(off, B)], WA[b], SM[b])
            if hk:
                pltpu.async_copy(wb_h.at[pl.ds(off, B)], WB[b], SM[b])

        def _drain_idx(n, b):
            off = base0 + n * B
            pltpu.make_async_copy(i0_h.at[pl.ds(off, B)], GI[b], SM[b]).wait()
            pltpu.make_async_copy(i1_h.at[pl.ds(off, B)], GJ[b], SM[b]).wait()
            pltpu.make_async_copy(wa_h.at[pl.ds(off, B)], WA[b], SM[b]).wait()
            if hk:
                pltpu.make_async_copy(wb_h.at[pl.ds(off, B)], WB[b], SM[b]).wait()

        def _gather(b):
            pltpu.async_copy(xn.at[GI[b]], XI[b], SG[b])
            pltpu.async_copy(xn.at[GJ[b]], XJ[b], SG[b])

        def _drain_gather(b):
            pltpu.make_async_copy(xn.at[GI[b]], XI[b], SG[b]).wait()
            pltpu.make_async_copy(xn.at[GJ[b]], XJ[b], SG[b]).wait()

        def _scatter(b):
            pltpu.async_copy(XJ[b], fk_acc.at[SI[b]], SS[b], add=True)
            pltpu.async_copy(XI[b], fk_acc.at[SJ[b]], SS[b], add=True)

        def _drain_scatter(b):
            pltpu.make_async_copy(XJ[b], fk_acc.at[SI[b]], SS[b]).wait()
            pltpu.make_async_copy(XI[b], fk_acc.at[SJ[b]], SS[b]).wait()

        def _compute(b):
            xi_b, xj_b = XI[b], XJ[b]
            wa_b, wb_b = WA[b], WB[b]
            gi_b, gj_b = GI[b], GJ[b]

            @plsc.parallel_loop(0, B, unroll=16)
            def _edge(e):
                xiv = [xi_b[e, pl.ds(k * 16, 16)] for k in range(D // 16)]
                xjv = [xj_b[e, pl.ds(k * 16, 16)] for k in range(D // 16)]
                acc = xiv[0] * xjv[0]
                for k in range(1, D // 16):
                    acc = acc + xiv[k] * xjv[k]
                sv = jnp.broadcast_to(jnp.sum(acc), (16,))
                if hk:
                    iiv = _splat(gi_b, e)
                    jjv = _splat(gj_b, e)
                    giv = plsc.load_gather(gall, [iiv])
                    gjv = plsc.load_gather(gall, [jjv])
                    cv = _splat(wb_b, e) * sv
                    plsc.addupdate_scatter(fh, [iiv], cv * gjv, mask=lane0)
                    plsc.addupdate_scatter(fh, [jjv], cv * giv, mask=lane0)
                    qv = -(giv * gjv * _splat(wa_b, e))
                else:
                    qv = -_splat(wa_b, e) * sv
                for k in range(D // 16):
                    xj_b[e, pl.ds(k * 16, 16)] = qv * xjv[k]
                    xi_b[e, pl.ds(k * 16, 16)] = qv * xiv[k]

        def _save_idx(b):
            for k in range(B // 16):
                s = pl.ds(k * 16, 16)
                SI[b][s] = GI[b][s]
                SJ[b][s] = GJ[b][s]

        _idx_dma(0, 0)
        _idx_dma(1, 1)
        _drain_idx(0, 0)
        _gather(0)

        @pl.loop(0, C, step=2)
        def _loop(c):
            for b in (0, 1):
                o = 1 - b
                n = c + b
                if b == 0:
                    @pl.when(c > 0)
                    def _():
                        _drain_scatter(o)
                    _drain_idx(n + 1, o)
                    _gather(o)
                else:
                    _drain_scatter(o)

                    @pl.when(c < C - 2)
                    def _():
                        _drain_idx(n + 1, o)
                        _gather(o)
                _drain_gather(b)
                _save_idx(b)
                _compute(b)

                @pl.when(c < C - 2)
                def _():
                    _idx_dma(n + 2, b)
                _scatter(b)

        _drain_scatter((C - 1) % 2)

    _phase(wid * (CK * B), CK, ik0, ik1, wk, wk, False)
    _phase(wid * (CHK * B), CHK, ih0, ih1, whk, whh, True)
    plsc.subcore_barrier()

    pltpu.sync_copy(fk_acc.at[pl.ds(sid * RPT, RPT)],
                    fko.at[cid, pl.ds(sid * RPT, RPT)])
    pltpu.sync_copy(fh, fho.at[wid])


def _post_body(Xn_ref, fkp_ref, fhp_ref, h0_ref, fH_ref, fK_ref):
    fH_ref[...] = h0_ref[...] + jnp.sum(fhp_ref[...], axis=0, keepdims=True)
    X = Xn_ref[...]
    fKp = fkp_ref[0] + fkp_ref[1]
    proj = jnp.sum(X * fKp, axis=1, keepdims=True)
    fK_ref[...] = -fKp + X * proj


def _post(Xn, fkp, fhp, h0):
    return pl.pallas_call(
        _post_body,
        out_shape=[
            jax.ShapeDtypeStruct((1, N), jnp.float32),
            jax.ShapeDtypeStruct((N, D), jnp.float32),
        ],
    )(Xn, fkp, fhp, h0)


def kernel(t, state_H, state_K, ind_K, ind_HK, kappa_K, kappa_H, weights_H, bias_H, weights_HK, weights_K):
    g2, Xn = _prea(state_H, state_K)
    (h0,) = _preb(state_H, bias_H, weights_H)
    g = g2.reshape(N)
    whh = weights_HK[:, 0] / kappa_H
    whk = weights_HK[:, 0] / kappa_K
    fkp, fhp = _edges(Xn, g, ind_K[:, 0], ind_K[:, 1], weights_K,
                      ind_HK[:, 0], ind_HK[:, 1], whk, whh)
    fH2, fK = _post(Xn, fkp, fhp, h0)
    return (fH2.reshape(N), fK)
